# baseline (speedup 1.0000x reference)
import dataclasses
import functools

import jax
import jax.numpy as jnp
from jax import lax
from jax.experimental import pallas as pl
from jax.experimental.pallas import tpu as pltpu
from jax.experimental.pallas import tpu_sc as plsc

_WINDOW = 128
_LANES = 16


@functools.partial(jax.jit, static_argnames=("n", "embed"))
def _sc_expand_rows(table, idx_flat, n, embed):
    vocab = table.shape[0]
    mesh = plsc.VectorSubcoreMesh(core_axis_name="core",
                                  subcore_axis_name="subcore")

    cp = pltpu.CompilerParams()
    if "needs_layout_passes" in pltpu.CompilerParams.__dataclass_fields__:
        cp = dataclasses.replace(cp, needs_layout_passes=False)

    @functools.partial(
        pl.kernel,
        out_type=jax.ShapeDtypeStruct((n, embed), table.dtype),
        mesh=mesh,
        scratch_types=[pltpu.VMEM((vocab, embed), table.dtype)],
        compiler_params=cp,
    )
    def expand_kernel(table_hbm, idx_hbm, out_hbm, table_v):
        pltpu.sync_copy(table_hbm, table_v)

        def body(i_vmem, o_vmem):
            @pl.loop(0, _WINDOW // _LANES)
            def _tile(t):
                lane = lax.iota(jnp.int32, _LANES) + t * _LANES
                row = i_vmem[0, pl.ds(t * _LANES, _LANES)]
                for c in range(embed):
                    colv = jnp.full((_LANES,), c, jnp.int32)
                    v = plsc.load_gather(table_v, [row, colv])
                    plsc.store_scatter(o_vmem, [lane, colv], v)

        pltpu.emit_pipeline(
            body,
            grid=(n // _WINDOW,),
            in_specs=[pl.BlockSpec((1, _WINDOW), index_map=lambda i: (0, i))],
            out_specs=[pl.BlockSpec((_WINDOW, embed),
                                    index_map=lambda i: (i, 0))],
            core_axis_name=("core", "subcore"),
            dimension_semantics=(pltpu.PARALLEL,),
        )(idx_hbm, out_hbm)

    return expand_kernel(table, idx_flat)


def kernel(seq_input, table):
    batch, seq = seq_input.shape
    vocab, embed = table.shape
    n = batch * seq
    idx_flat = seq_input.reshape(1, n).astype(jnp.int32)
    out = _sc_expand_rows(table, idx_flat, n, embed)
    return out.reshape(batch, seq, embed)


# speedup vs baseline: 4.5609x; 4.5609x over previous
import dataclasses
import functools

import jax
import jax.numpy as jnp
from jax import lax
from jax.experimental import pallas as pl
from jax.experimental.pallas import tpu as pltpu
from jax.experimental.pallas import tpu_sc as plsc

_WINDOW = 128
_LANES = 16


@functools.partial(jax.jit, static_argnames=("n", "vocab", "embed"))
def _sc_expand_rows(table_flat, idx_flat, n, vocab, embed):
    mesh = plsc.VectorSubcoreMesh(core_axis_name="core",
                                  subcore_axis_name="subcore")
    info = plsc.get_sparse_core_info()
    nworkers = info.num_cores * info.num_subcores
    bpw = n // (_WINDOW * nworkers)
    assert bpw % 2 == 0

    cp = pltpu.CompilerParams()
    if "needs_layout_passes" in pltpu.CompilerParams.__dataclass_fields__:
        cp = dataclasses.replace(cp, needs_layout_passes=False)

    @functools.partial(
        pl.kernel,
        out_type=jax.ShapeDtypeStruct((n, embed), table_flat.dtype),
        mesh=mesh,
        scratch_types=[
            pltpu.VMEM((vocab * embed,), table_flat.dtype),
            pltpu.VMEM((_WINDOW,), jnp.int32),
            pltpu.VMEM((_WINDOW, embed), table_flat.dtype),
            pltpu.VMEM((_WINDOW, embed), table_flat.dtype),
            pltpu.SemaphoreType.DMA,
            pltpu.SemaphoreType.DMA,
        ],
        compiler_params=cp,
    )
    def expand_kernel(table_hbm, idx_hbm, out_hbm, table_v, i_v,
                      obuf0, obuf1, sem0, sem1):
        pltpu.sync_copy(table_hbm, table_v)
        wid = (lax.axis_index("subcore") * info.num_cores
               + lax.axis_index("core"))
        base = wid * bpw * _WINDOW
        iota = lax.iota(jnp.int32, _LANES)

        def expand_block(step, obuf, sem):
            row0 = base + step * _WINDOW
            pltpu.sync_copy(idx_hbm.at[0, pl.ds(row0, _WINDOW)], i_v)

            @pl.loop(0, _WINDOW // _LANES)
            def _group(g):
                idx16 = i_v[pl.ds(g * _LANES, _LANES)]
                for j2 in range(_LANES):
                    rv = lax.gather(
                        idx16,
                        jnp.full((_LANES, 1), j2, jnp.int32),
                        lax.GatherDimensionNumbers(
                            offset_dims=(),
                            collapsed_slice_dims=(0,),
                            start_index_map=(0,)),
                        slice_sizes=(1,),
                        mode=lax.GatherScatterMode.PROMISE_IN_BOUNDS)
                    rbase = rv * embed
                    j = g * _LANES + j2
                    for k in range(embed // _LANES):
                        src = rbase + (iota + k * _LANES)
                        v = plsc.load_gather(table_v, [src])
                        obuf[j, pl.ds(k * _LANES, _LANES)] = v

            @pl.when(step >= 2)
            def _():
                pltpu.make_async_copy(
                    obuf, out_hbm.at[pl.ds(row0, _WINDOW)], sem).wait()

            pltpu.async_copy(obuf, out_hbm.at[pl.ds(row0, _WINDOW)], sem)

        @pl.loop(0, bpw, step=2)
        def _pair(step):
            expand_block(step, obuf0, sem0)
            expand_block(step + 1, obuf1, sem1)

        pltpu.make_async_copy(obuf0, out_hbm.at[pl.ds(base, _WINDOW)],
                              sem0).wait()
        pltpu.make_async_copy(obuf1, out_hbm.at[pl.ds(base, _WINDOW)],
                              sem1).wait()

    return expand_kernel(table_flat, idx_flat)


def kernel(seq_input, table):
    batch, seq = seq_input.shape
    vocab, embed = table.shape
    n = batch * seq
    idx_flat = seq_input.reshape(1, n).astype(jnp.int32)
    out = _sc_expand_rows(table.reshape(-1), idx_flat, n, vocab, embed)
    return out.reshape(batch, seq, embed)


# speedup vs baseline: 9.3181x; 2.0430x over previous
import functools

import jax
import jax.numpy as jnp
from jax import lax
from jax.experimental import pallas as pl
from jax.experimental.pallas import tpu as pltpu
from jax.experimental.pallas import tpu_sc as plsc

_WINDOW = 128
_REPLICAS = 64

_TC_BLK = 1024
_VPAD = 32

_N_SC = 81920


@functools.partial(jax.jit, static_argnames=("n", "n_sc", "embed"))
def _sc_gather_rows(table_rep, idx_sc, n, n_sc, embed):
    mesh = plsc.VectorSubcoreMesh(core_axis_name="core",
                                  subcore_axis_name="subcore")

    @functools.partial(
        pl.kernel,
        out_type=jax.ShapeDtypeStruct((n, embed), table_rep.dtype),
        mesh=mesh,
    )
    def gather_kernel(table_hbm, idx_hbm, out_hbm):
        def body(i_vmem, o_vmem):
            pltpu.sync_copy(table_hbm.at[i_vmem.at[0]], o_vmem)

        pltpu.emit_pipeline(
            body,
            grid=(n_sc // _WINDOW,),
            in_specs=[pl.BlockSpec((1, _WINDOW), index_map=lambda i: (0, i))],
            out_specs=[pl.BlockSpec((_WINDOW, embed),
                                    index_map=lambda i: (i, 0))],
            core_axis_name=("core", "subcore"),
            dimension_semantics=(pltpu.PARALLEL,),
        )(idx_hbm, out_hbm)

    return gather_kernel(table_rep, idx_sc)


def _tc_onehot_kernel(idx_ref, w_ref, buf_ref, out_ref):
    del buf_ref
    idx = idx_ref[0, 0, :]
    k_iota = lax.broadcasted_iota(jnp.int32, (_TC_BLK, 2 * _VPAD), 1)
    onehot = (jnp.bitwise_and(k_iota, _VPAD - 1) == idx[:, None])
    out_ref[...] = jnp.dot(onehot.astype(jnp.bfloat16), w_ref[...],
                           preferred_element_type=jnp.float32)


@functools.partial(jax.jit, static_argnames=("n", "n_sc", "embed"))
def _tc_onehot_rows(w_hi_lo, idx_tc, buf, n, n_sc, embed):
    nblk = (n - n_sc) // _TC_BLK
    blk0 = n_sc // _TC_BLK
    idx3 = idx_tc.reshape(nblk, 1, _TC_BLK)
    return pl.pallas_call(
        _tc_onehot_kernel,
        grid=(nblk,),
        in_specs=[
            pl.BlockSpec((1, 1, _TC_BLK), lambda i: (i, 0, 0)),
            pl.BlockSpec((2 * _VPAD, embed), lambda i: (0, 0)),
            pl.BlockSpec(memory_space=pl.ANY),
        ],
        out_specs=pl.BlockSpec((_TC_BLK, embed), lambda i: (i + blk0, 0)),
        out_shape=jax.ShapeDtypeStruct((n, embed), jnp.float32),
        input_output_aliases={2: 0},
    )(idx3, w_hi_lo, buf)


def _trunc_bf16(x):
    u = lax.bitcast_convert_type(x, jnp.uint32)
    head_f = lax.bitcast_convert_type(
        jnp.bitwise_and(u, jnp.uint32(0xFFFF0000)), jnp.float32)
    head_bf = lax.bitcast_convert_type(
        (u >> 16).astype(jnp.uint16), jnp.bfloat16)
    return head_bf, x - head_f


def _make_hi_lo(table, vocab, embed):
    tpad = jnp.zeros((_VPAD, embed), table.dtype).at[:vocab].set(table)
    hi_bf, resid = _trunc_bf16(tpad)
    lo_bf, _ = _trunc_bf16(resid)
    return jnp.concatenate([hi_bf, lo_bf], axis=0)


def kernel(seq_input, table):
    batch, seq = seq_input.shape
    vocab, embed = table.shape
    n = batch * seq
    idx_flat = seq_input.reshape(n).astype(jnp.int32)
    n_sc = _N_SC

    table_rep = jnp.tile(table, (_REPLICAS, 1))
    nblk_sc = n_sc // _WINDOW
    block_off = (jnp.arange(nblk_sc, dtype=jnp.int32) % _REPLICAS) * vocab
    idx_sc = (idx_flat[:n_sc].reshape(nblk_sc, _WINDOW)
              + block_off[:, None]).reshape(1, n_sc)

    w_hi_lo = _make_hi_lo(table, vocab, embed)

    buf = _sc_gather_rows(table_rep, idx_sc, n, n_sc, embed)
    out = _tc_onehot_rows(w_hi_lo, idx_flat[n_sc:], buf, n, n_sc, embed)
    return out.reshape(batch, seq, embed)


# speedup vs baseline: 11.0018x; 1.1807x over previous
import functools

import jax
import jax.numpy as jnp
from jax import lax
from jax.experimental import pallas as pl
from jax.experimental.pallas import tpu as pltpu
from jax.experimental.pallas import tpu_sc as plsc

_WINDOW = 128
_REPLICAS = 64

_TC_BLK = 2048
_VPAD = 32

_N_SC = 81920


@functools.partial(jax.jit, static_argnames=("n", "n_sc", "embed"))
def _sc_gather_rows(table_rep, idx_sc, n, n_sc, embed):
    mesh = plsc.VectorSubcoreMesh(core_axis_name="core",
                                  subcore_axis_name="subcore")

    @functools.partial(
        pl.kernel,
        out_type=jax.ShapeDtypeStruct((n, embed), table_rep.dtype),
        mesh=mesh,
    )
    def gather_kernel(table_hbm, idx_hbm, out_hbm):
        def body(i_vmem, o_vmem):
            pltpu.sync_copy(table_hbm.at[i_vmem.at[0]], o_vmem)

        pltpu.emit_pipeline(
            body,
            grid=(n_sc // _WINDOW,),
            in_specs=[pl.BlockSpec((1, _WINDOW), index_map=lambda i: (0, i))],
            out_specs=[pl.BlockSpec((_WINDOW, embed),
                                    index_map=lambda i: (i, 0))],
            core_axis_name=("core", "subcore"),
            dimension_semantics=(pltpu.PARALLEL,),
        )(idx_hbm, out_hbm)

    return gather_kernel(table_rep, idx_sc)


def _tc_onehot_kernel(idx_ref, w_ref, buf_ref, out_ref):
    del buf_ref
    idx = idx_ref[0, 0, :]
    k_iota = lax.broadcasted_iota(jnp.int32, (_TC_BLK, 2 * _VPAD), 1)
    onehot = (jnp.bitwise_and(k_iota, _VPAD - 1) == idx[:, None])
    out_ref[...] = jnp.dot(onehot.astype(jnp.bfloat16), w_ref[...],
                           preferred_element_type=jnp.float32)


@functools.partial(jax.jit, static_argnames=("n", "n_sc", "embed"))
def _tc_onehot_rows(w_hi_lo, idx_tc, buf, n, n_sc, embed):
    nblk = (n - n_sc) // _TC_BLK
    blk0 = n_sc // _TC_BLK
    idx3 = idx_tc.reshape(nblk, 1, _TC_BLK)
    return pl.pallas_call(
        _tc_onehot_kernel,
        grid=(nblk,),
        in_specs=[
            pl.BlockSpec((1, 1, _TC_BLK), lambda i: (i, 0, 0)),
            pl.BlockSpec((2 * _VPAD, embed), lambda i: (0, 0)),
            pl.BlockSpec(memory_space=pl.ANY),
        ],
        out_specs=pl.BlockSpec((_TC_BLK, embed), lambda i: (i + blk0, 0)),
        out_shape=jax.ShapeDtypeStruct((n, embed), jnp.float32),
        input_output_aliases={2: 0},
    )(idx3, w_hi_lo, buf)


def _trunc_bf16(x):
    u = lax.bitcast_convert_type(x, jnp.uint32)
    head_f = lax.bitcast_convert_type(
        jnp.bitwise_and(u, jnp.uint32(0xFFFF0000)), jnp.float32)
    head_bf = lax.bitcast_convert_type(
        (u >> 16).astype(jnp.uint16), jnp.bfloat16)
    return head_bf, x - head_f


def _make_hi_lo(table, vocab, embed):
    tpad = jnp.zeros((_VPAD, embed), table.dtype).at[:vocab].set(table)
    hi_bf, resid = _trunc_bf16(tpad)
    lo_bf, _ = _trunc_bf16(resid)
    return jnp.concatenate([hi_bf, lo_bf], axis=0)


def kernel(seq_input, table):
    batch, seq = seq_input.shape
    vocab, embed = table.shape
    n = batch * seq
    idx_flat = seq_input.reshape(n).astype(jnp.int32)
    n_sc = _N_SC

    table_rep = jnp.tile(table, (_REPLICAS, 1))
    nblk_sc = n_sc // _WINDOW
    block_off = (jnp.arange(nblk_sc, dtype=jnp.int32) % _REPLICAS) * vocab
    idx_sc = (idx_flat[:n_sc].reshape(nblk_sc, _WINDOW)
              + block_off[:, None]).reshape(1, n_sc)

    w_hi_lo = _make_hi_lo(table, vocab, embed)

    buf = _sc_gather_rows(table_rep, idx_sc, n, n_sc, embed)
    out = _tc_onehot_rows(w_hi_lo, idx_flat[n_sc:], buf, n, n_sc, embed)
    return out.reshape(batch, seq, embed)


# speedup vs baseline: 12.1479x; 1.1042x over previous
import functools

import jax
import jax.numpy as jnp
from jax import lax
from jax.experimental import pallas as pl
from jax.experimental.pallas import tpu as pltpu
from jax.experimental.pallas import tpu_sc as plsc

_WINDOW = 128
_REPLICAS = 64

_TC_BLK = 4096
_VPAD = 32

_N_SC = 81920


@functools.partial(jax.jit, static_argnames=("n", "n_sc", "embed"))
def _sc_gather_rows(table_rep, idx_sc, n, n_sc, embed):
    mesh = plsc.VectorSubcoreMesh(core_axis_name="core",
                                  subcore_axis_name="subcore")

    @functools.partial(
        pl.kernel,
        out_type=jax.ShapeDtypeStruct((n, embed), table_rep.dtype),
        mesh=mesh,
    )
    def gather_kernel(table_hbm, idx_hbm, out_hbm):
        def body(i_vmem, o_vmem):
            pltpu.sync_copy(table_hbm.at[i_vmem.at[0]], o_vmem)

        pltpu.emit_pipeline(
            body,
            grid=(n_sc // _WINDOW,),
            in_specs=[pl.BlockSpec((1, _WINDOW), index_map=lambda i: (0, i))],
            out_specs=[pl.BlockSpec((_WINDOW, embed),
                                    index_map=lambda i: (i, 0))],
            core_axis_name=("core", "subcore"),
            dimension_semantics=(pltpu.PARALLEL,),
        )(idx_hbm, out_hbm)

    return gather_kernel(table_rep, idx_sc)


def _tc_onehot_kernel(idx_ref, w_ref, buf_ref, out_ref):
    del buf_ref
    idx = idx_ref[0, 0, :]
    k_iota = lax.broadcasted_iota(jnp.int32, (_TC_BLK, 2 * _VPAD), 1)
    onehot = (jnp.bitwise_and(k_iota, _VPAD - 1) == idx[:, None])
    out_ref[...] = jnp.dot(onehot.astype(jnp.bfloat16), w_ref[...],
                           preferred_element_type=jnp.float32)


@functools.partial(jax.jit, static_argnames=("n", "n_sc", "embed"))
def _tc_onehot_rows(w_hi_lo, idx_tc, buf, n, n_sc, embed):
    nblk = (n - n_sc) // _TC_BLK
    blk0 = n_sc // _TC_BLK
    idx3 = idx_tc.reshape(nblk, 1, _TC_BLK)
    return pl.pallas_call(
        _tc_onehot_kernel,
        grid=(nblk,),
        in_specs=[
            pl.BlockSpec((1, 1, _TC_BLK), lambda i: (i, 0, 0)),
            pl.BlockSpec((2 * _VPAD, embed), lambda i: (0, 0)),
            pl.BlockSpec(memory_space=pl.ANY),
        ],
        out_specs=pl.BlockSpec((_TC_BLK, embed), lambda i: (i + blk0, 0)),
        out_shape=jax.ShapeDtypeStruct((n, embed), jnp.float32),
        input_output_aliases={2: 0},
    )(idx3, w_hi_lo, buf)


def _trunc_bf16(x):
    u = lax.bitcast_convert_type(x, jnp.uint32)
    head_f = lax.bitcast_convert_type(
        jnp.bitwise_and(u, jnp.uint32(0xFFFF0000)), jnp.float32)
    head_bf = lax.bitcast_convert_type(
        (u >> 16).astype(jnp.uint16), jnp.bfloat16)
    return head_bf, x - head_f


def _make_hi_lo(table, vocab, embed):
    tpad = jnp.zeros((_VPAD, embed), table.dtype).at[:vocab].set(table)
    hi_bf, resid = _trunc_bf16(tpad)
    lo_bf, _ = _trunc_bf16(resid)
    return jnp.concatenate([hi_bf, lo_bf], axis=0)


def kernel(seq_input, table):
    batch, seq = seq_input.shape
    vocab, embed = table.shape
    n = batch * seq
    idx_flat = seq_input.reshape(n).astype(jnp.int32)
    n_sc = _N_SC

    table_rep = jnp.tile(table, (_REPLICAS, 1))
    nblk_sc = n_sc // _WINDOW
    block_off = (jnp.arange(nblk_sc, dtype=jnp.int32) % _REPLICAS) * vocab
    idx_sc = (idx_flat[:n_sc].reshape(nblk_sc, _WINDOW)
              + block_off[:, None]).reshape(1, n_sc)

    w_hi_lo = _make_hi_lo(table, vocab, embed)

    buf = _sc_gather_rows(table_rep, idx_sc, n, n_sc, embed)
    out = _tc_onehot_rows(w_hi_lo, idx_flat[n_sc:], buf, n, n_sc, embed)
    return out.reshape(batch, seq, embed)


# speedup vs baseline: 14.1839x; 1.1676x over previous
import functools

import jax
import jax.numpy as jnp
from jax import lax
from jax.experimental import pallas as pl
from jax.experimental.pallas import tpu as pltpu
from jax.experimental.pallas import tpu_sc as plsc

_WINDOW = 128
_REPLICAS = 64

_TC_BLK = 4096
_VPAD = 32

_N_SC = 53248


@functools.partial(jax.jit, static_argnames=("n", "n_sc", "embed"))
def _sc_gather_rows(table_rep, idx_sc, n, n_sc, embed):
    mesh = plsc.VectorSubcoreMesh(core_axis_name="core",
                                  subcore_axis_name="subcore")

    @functools.partial(
        pl.kernel,
        out_type=jax.ShapeDtypeStruct((n, embed), table_rep.dtype),
        mesh=mesh,
    )
    def gather_kernel(table_hbm, idx_hbm, out_hbm):
        def body(i_vmem, o_vmem):
            pltpu.sync_copy(table_hbm.at[i_vmem.at[0]], o_vmem)

        pltpu.emit_pipeline(
            body,
            grid=(n_sc // _WINDOW,),
            in_specs=[pl.BlockSpec((1, _WINDOW), index_map=lambda i: (0, i))],
            out_specs=[pl.BlockSpec((_WINDOW, embed),
                                    index_map=lambda i: (i, 0))],
            core_axis_name=("core", "subcore"),
            dimension_semantics=(pltpu.PARALLEL,),
        )(idx_hbm, out_hbm)

    return gather_kernel(table_rep, idx_sc)


def _tc_onehot_kernel(idx_ref, w_ref, buf_ref, out_ref):
    del buf_ref
    idx = idx_ref[0, 0, :]
    k_iota = lax.broadcasted_iota(jnp.int32, (_TC_BLK, 2 * _VPAD), 1)
    onehot = (jnp.bitwise_and(k_iota, _VPAD - 1) == idx[:, None])
    out_ref[...] = jnp.dot(onehot.astype(jnp.bfloat16), w_ref[...],
                           preferred_element_type=jnp.float32)


@functools.partial(jax.jit, static_argnames=("n", "n_sc", "embed"))
def _tc_onehot_rows(w_hi_lo, idx_tc, buf, n, n_sc, embed):
    nblk = (n - n_sc) // _TC_BLK
    blk0 = n_sc // _TC_BLK
    idx3 = idx_tc.reshape(nblk, 1, _TC_BLK)
    return pl.pallas_call(
        _tc_onehot_kernel,
        grid=(nblk,),
        in_specs=[
            pl.BlockSpec((1, 1, _TC_BLK), lambda i: (i, 0, 0)),
            pl.BlockSpec((2 * _VPAD, embed), lambda i: (0, 0)),
            pl.BlockSpec(memory_space=pl.ANY),
        ],
        out_specs=pl.BlockSpec((_TC_BLK, embed), lambda i: (i + blk0, 0)),
        out_shape=jax.ShapeDtypeStruct((n, embed), jnp.float32),
        input_output_aliases={2: 0},
    )(idx3, w_hi_lo, buf)


def _trunc_bf16(x):
    u = lax.bitcast_convert_type(x, jnp.uint32)
    head_f = lax.bitcast_convert_type(
        jnp.bitwise_and(u, jnp.uint32(0xFFFF0000)), jnp.float32)
    head_bf = lax.bitcast_convert_type(
        (u >> 16).astype(jnp.uint16), jnp.bfloat16)
    return head_bf, x - head_f


def _make_hi_lo(table, vocab, embed):
    tpad = jnp.zeros((_VPAD, embed), table.dtype).at[:vocab].set(table)
    hi_bf, resid = _trunc_bf16(tpad)
    lo_bf, _ = _trunc_bf16(resid)
    return jnp.concatenate([hi_bf, lo_bf], axis=0)


def kernel(seq_input, table):
    batch, seq = seq_input.shape
    vocab, embed = table.shape
    n = batch * seq
    idx_flat = seq_input.reshape(n).astype(jnp.int32)
    n_sc = _N_SC

    table_rep = jnp.tile(table, (_REPLICAS, 1))
    nblk_sc = n_sc // _WINDOW
    block_off = (jnp.arange(nblk_sc, dtype=jnp.int32) % _REPLICAS) * vocab
    idx_sc = (idx_flat[:n_sc].reshape(nblk_sc, _WINDOW)
              + block_off[:, None]).reshape(1, n_sc)

    w_hi_lo = _make_hi_lo(table, vocab, embed)

    buf = _sc_gather_rows(table_rep, idx_sc, n, n_sc, embed)
    out = _tc_onehot_rows(w_hi_lo, idx_flat[n_sc:], buf, n, n_sc, embed)
    return out.reshape(batch, seq, embed)
